# baseline (speedup 1.0000x reference)
import jax
import jax.numpy as jnp
from jax import lax
from jax.experimental import pallas as pl
from jax.experimental.pallas import tpu as pltpu
from jax.experimental.pallas import tpu_sc as plsc

NC = 2
NS = 16
NW = NC * NS

NSLOT = 5


def _make_sc_call(B, L, N, D, V):
    gpw = (B * L) // NW
    wpb = NW // B
    lpw = L // wpb
    assert gpw * NW == B * L and lpw * wpb == L and lpw == gpw

    mesh = plsc.VectorSubcoreMesh(core_axis_name="c", subcore_axis_name="s")

    nblk = gpw // NSLOT
    assert nblk * NSLOT == gpw

    def body(x_hbm, idx_hbm, pe_hbm, out_hbm, idx_v, bufs, pe_sh,
             sem_in, sem_g, sem_out):
        wid = lax.axis_index("s") * NC + lax.axis_index("c")
        s = lax.axis_index("s")
        b = wid // wpb
        b8 = (b // 8) * 8
        l0 = (wid % wpb) * lpw
        nst = 8
        chunk = (-(-V // nst) + 7) // 8 * 8
        for t in range(nst):
            lo = t * chunk
            sz = min(chunk, V - lo)
            if sz <= 0:
                break

            @pl.when(s == t)
            def _(lo=lo, sz=sz):
                pltpu.sync_copy(pe_hbm.at[pl.ds(lo, sz)],
                                pe_sh.at[pl.ds(lo, sz)])
        h_in = [None] * nblk
        h_out = [None] * nblk

        def s1(g):
            h_in[g] = pltpu.async_copy(
                x_hbm.at[b, pl.ds(l0 + g * NSLOT, NSLOT)], bufs.at[g % 2],
                sem_in.at[g % 2])

        s1(0)
        if nblk > 1:
            s1(1)
        pltpu.sync_copy(idx_hbm.at[pl.ds(l0, lpw), pl.ds(b8, 8)], idx_v)
        plsc.subcore_barrier()

        for g in range(nblk):
            if g + 1 < nblk and g >= 1:
                for h in h_out[g - 1]:
                    h.wait()
                s1(g + 1)
            h_in[g].wait()
            h_g = [pltpu.async_copy(
                pe_sh.at[idx_v.at[g * NSLOT + k, b - b8]],
                bufs.at[g % 2, k], sem_g.at[g % 2, k], add=True)
                for k in range(NSLOT)]
            h_out[g] = []
            for k in range(NSLOT):
                h_g[k].wait()
                h_out[g].append(pltpu.async_copy(
                    bufs.at[g % 2, k], out_hbm.at[b, l0 + g * NSLOT + k],
                    sem_out.at[g % 2]))
        for g in (nblk - 2, nblk - 1):
            for h in h_out[g]:
                h.wait()

    return pl.kernel(
        body,
        out_type=jax.ShapeDtypeStruct((B, L, N, D), jnp.float32),
        mesh=mesh,
        scratch_types=[
            pltpu.VMEM((lpw, 8, N), jnp.int32),
            pltpu.VMEM((2, NSLOT, N, D), jnp.float32),
            pltpu.VMEM_SHARED((V, D), jnp.float32),
            pltpu.SemaphoreType.DMA((2,)),
            pltpu.SemaphoreType.DMA((2, NSLOT)),
            pltpu.SemaphoreType.DMA((2,)),
        ],
    )


def kernel(input_emb, position, pe):
    B, N, L, D = input_emb.shape
    x = input_emb.transpose(0, 2, 1, 3)
    idx = position.transpose(2, 0, 1).astype(jnp.int32)
    out = _make_sc_call(B, L, N, D, pe.shape[0])(x, idx, pe)
    return out.transpose(0, 2, 1, 3)


# speedup vs baseline: 1.0322x; 1.0322x over previous
import jax
import jax.numpy as jnp
from jax import lax
from jax.experimental import pallas as pl
from jax.experimental.pallas import tpu as pltpu
from jax.experimental.pallas import tpu_sc as plsc

NC = 2
NS = 16
NW = NC * NS

NSLOT = 5


def _make_sc_call(B, L, N, D, V):
    gpw = (B * L) // NW
    wpb = NW // B
    lpw = L // wpb
    assert gpw * NW == B * L and lpw * wpb == L and lpw == gpw

    mesh = plsc.VectorSubcoreMesh(core_axis_name="c", subcore_axis_name="s")

    nblk = gpw // NSLOT
    assert nblk * NSLOT == gpw

    def body(x_hbm, idx_hbm, pe_hbm, out_hbm, idx_v, bufs, pe_sh,
             sem_in, sem_g, sem_out):
        wid = lax.axis_index("s") * NC + lax.axis_index("c")
        s = lax.axis_index("s")
        b = wid // wpb
        b8 = (b // 8) * 8
        l0 = (wid % wpb) * lpw
        nst = 8
        chunk = (-(-V // nst) + 7) // 8 * 8
        for t in range(nst):
            lo = t * chunk
            sz = min(chunk, V - lo)
            if sz <= 0:
                break

            @pl.when(s == t)
            def _(lo=lo, sz=sz):
                pltpu.sync_copy(pe_hbm.at[pl.ds(lo, sz)],
                                pe_sh.at[pl.ds(lo, sz)])
        h_in = [None] * nblk
        h_out = [None] * nblk

        def s1(g):
            h_in[g] = pltpu.async_copy(
                x_hbm.at[b, pl.ds(l0 + g * NSLOT, NSLOT)], bufs.at[g % 2],
                sem_in.at[g % 2])

        s1(0)
        pltpu.sync_copy(idx_hbm.at[pl.ds(l0, lpw), pl.ds(b8, 8)], idx_v)
        plsc.subcore_barrier()

        for g in range(nblk):
            if g + 1 < nblk:
                if g >= 1:
                    for h in h_out[g - 1]:
                        h.wait()
                s1(g + 1)
            h_in[g].wait()
            h_g = [pltpu.async_copy(
                pe_sh.at[idx_v.at[g * NSLOT + k, b - b8]],
                bufs.at[g % 2, k], sem_g.at[g % 2, k], add=True)
                for k in range(NSLOT)]
            h_out[g] = []
            for k in range(NSLOT):
                h_g[k].wait()
                h_out[g].append(pltpu.async_copy(
                    bufs.at[g % 2, k], out_hbm.at[b, l0 + g * NSLOT + k],
                    sem_out.at[g % 2]))
        for g in (nblk - 2, nblk - 1):
            for h in h_out[g]:
                h.wait()

    return pl.kernel(
        body,
        out_type=jax.ShapeDtypeStruct((B, L, N, D), jnp.float32),
        mesh=mesh,
        scratch_types=[
            pltpu.VMEM((lpw, 8, N), jnp.int32),
            pltpu.VMEM((2, NSLOT, N, D), jnp.float32),
            pltpu.VMEM_SHARED((V, D), jnp.float32),
            pltpu.SemaphoreType.DMA((2,)),
            pltpu.SemaphoreType.DMA((2, NSLOT)),
            pltpu.SemaphoreType.DMA((2,)),
        ],
    )


def kernel(input_emb, position, pe):
    B, N, L, D = input_emb.shape
    x = input_emb.transpose(0, 2, 1, 3)
    idx = position.transpose(2, 0, 1).astype(jnp.int32)
    out = _make_sc_call(B, L, N, D, pe.shape[0])(x, idx, pe)
    return out.transpose(0, 2, 1, 3)
